# baseline (speedup 1.0000x reference)
import functools

import jax
import jax.numpy as jnp
from jax import lax
from jax.experimental import pallas as pl
from jax.experimental.pallas import tpu as pltpu
from jax.experimental.pallas import tpu_sc as plsc

_NBUF = 4
_K = 2


def _embed_body(nchunk, chunk, b_per_w, x_hbm, table_hbm, out_hbm,
                idx_v, *bufs_and_sems):
    rows = bufs_and_sems[:_NBUF]
    gsem = bufs_and_sems[_NBUF:2 * _NBUF]
    wsem = bufs_and_sems[2 * _NBUF:3 * _NBUF]

    wid = lax.axis_index("s") * 2 + lax.axis_index("c")
    base = wid * b_per_w
    pltpu.sync_copy(x_hbm.at[pl.ds(base, b_per_w)], idx_v)

    def start_gather(g, b):
        pltpu.async_copy(
            table_hbm.at[idx_v.at[pl.ds(g * chunk, chunk)]],
            rows[b], gsem[b])

    def wait_gather(g, b):
        pltpu.make_async_copy(
            table_hbm.at[idx_v.at[pl.ds(g * chunk, chunk)]],
            rows[b], gsem[b]).wait()

    def start_write(g, b):
        pltpu.async_copy(
            rows[b], out_hbm.at[pl.ds(base + g * chunk, chunk)], wsem[b])

    def wait_write(g, b):
        pltpu.make_async_copy(
            rows[b], out_hbm.at[pl.ds(base + g * chunk, chunk)],
            wsem[b]).wait()

    for b in range(_K):
        start_gather(b, b)

    ngroups = nchunk // _NBUF

    def group(go, _):
        for b in range(_NBUF):
            g = go * _NBUF + b
            nxt = (b + _K) % _NBUF
            @pl.when((g + _K < nchunk) & (g >= _NBUF - _K))
            def _():
                wait_write(g + _K - _NBUF, nxt)

            @pl.when(g + _K < nchunk)
            def _():
                start_gather(g + _K, nxt)

            wait_gather(g, b)
            start_write(g, b)
        return 0

    lax.fori_loop(0, ngroups, group, 0)

    for b in range(_NBUF):
        wait_write(nchunk - _NBUF + b, b)


@functools.partial(jax.jit, static_argnames=("b_total", "d", "chunk"))
def _embed(xt_flat, table, b_total, d, chunk):
    info = plsc.get_sparse_core_info()
    nw = info.num_cores * info.num_subcores
    b_per_w = b_total // nw
    nchunk = b_per_w // chunk
    mesh = plsc.VectorSubcoreMesh(core_axis_name="c", subcore_axis_name="s")
    kfn = pl.kernel(
        functools.partial(_embed_body, nchunk, chunk, b_per_w),
        mesh=mesh,
        out_type=jax.ShapeDtypeStruct((b_total, d), jnp.float32),
        scratch_types=(
            [pltpu.VMEM((b_per_w,), jnp.int32)]
            + [pltpu.VMEM((chunk, d), jnp.float32) for _ in range(_NBUF)]
            + [pltpu.SemaphoreType.DMA for _ in range(2 * _NBUF)]
        ),
    )
    return kfn(xt_flat, table)


def kernel(x, table):
    nbatch, seq = x.shape
    d = table.shape[1]
    xt_flat = jnp.reshape(jnp.transpose(x), (nbatch * seq,)).astype(jnp.int32)
    out = _embed(xt_flat, table, nbatch * seq, d, 200)
    return jnp.transpose(jnp.reshape(out, (seq, nbatch, d)), (1, 0, 2))


# speedup vs baseline: 1.0262x; 1.0262x over previous
import functools

import jax
import jax.numpy as jnp
from jax import lax
from jax.experimental import pallas as pl
from jax.experimental.pallas import tpu as pltpu
from jax.experimental.pallas import tpu_sc as plsc

_NBUF = 5
_K = 3


def _embed_body(seq, nb, nbatch, xt_hbm, table_hbm, out_hbm,
                idx_v, *bufs_and_sems):
    rows = bufs_and_sems[:_NBUF]
    gsem = bufs_and_sems[_NBUF:2 * _NBUF]
    wsem = bufs_and_sems[2 * _NBUF:3 * _NBUF]

    wid = lax.axis_index("s") * 2 + lax.axis_index("c")
    base = wid * nb
    pltpu.sync_copy(xt_hbm.at[:, pl.ds(base, nb)], idx_v)

    def start_gather(g, b):
        pltpu.async_copy(table_hbm.at[idx_v.at[g]], rows[b], gsem[b])

    def wait_gather(g, b):
        pltpu.make_async_copy(
            table_hbm.at[idx_v.at[g]], rows[b], gsem[b]).wait()

    def start_write(g, b):
        pltpu.async_copy(
            rows[b], out_hbm.at[pl.ds(g * nbatch + base, nb)], wsem[b])

    def wait_write(g, b):
        pltpu.make_async_copy(
            rows[b], out_hbm.at[pl.ds(g * nbatch + base, nb)],
            wsem[b]).wait()

    for b in range(_K):
        start_gather(b, b)

    ngroups = seq // _NBUF

    def group(go, _):
        for b in range(_NBUF):
            g = go * _NBUF + b
            nxt = (b + _K) % _NBUF
            @pl.when((g + _K < seq) & (g >= _NBUF - _K))
            def _():
                wait_write(g + _K - _NBUF, nxt)

            @pl.when(g + _K < seq)
            def _():
                start_gather(g + _K, nxt)

            wait_gather(g, b)
            start_write(g, b)
        return 0

    lax.fori_loop(0, ngroups, group, 0)

    for b in range(_NBUF):
        wait_write(seq - _NBUF + b, b)


@functools.partial(jax.jit, static_argnames=("nbatch", "seq", "d"))
def _embed(xt, table, nbatch, seq, d):
    info = plsc.get_sparse_core_info()
    nw = info.num_cores * info.num_subcores
    nb = nbatch // nw
    mesh = plsc.VectorSubcoreMesh(core_axis_name="c", subcore_axis_name="s")
    kfn = pl.kernel(
        functools.partial(_embed_body, seq, nb, nbatch),
        mesh=mesh,
        out_type=jax.ShapeDtypeStruct((seq * nbatch, d), jnp.float32),
        scratch_types=(
            [pltpu.VMEM((seq, nb), jnp.int32)]
            + [pltpu.VMEM((nb, d), jnp.float32) for _ in range(_NBUF)]
            + [pltpu.SemaphoreType.DMA for _ in range(2 * _NBUF)]
        ),
    )
    return kfn(xt, table)


def kernel(x, table):
    nbatch, seq = x.shape
    d = table.shape[1]
    xt = jnp.transpose(x).astype(jnp.int32)
    out = _embed(xt, table, nbatch, seq, d)
    return jnp.transpose(jnp.reshape(out, (seq, nbatch, d)), (1, 0, 2))
